# baseline (speedup 1.0000x reference)
import functools

import jax
import jax.numpy as jnp
from jax import lax
from jax.experimental import pallas as pl
from jax.experimental.pallas import tpu as pltpu
from jax.experimental.pallas import tpu_sc as plsc

PAD = 0
BATCH = 4096
SEQ = 200
D = 128
PE_ROWS = 201
NC, NS, L = 2, 16, 16
NW = NC * NS
RPW = BATCH // NW
ROW_WORDS = SEQ * D
PE_WORDS = PE_ROWS * D
NWIN = SEQ // L
STAGE_K = L - 1
G = 64


def _bcast_body(clean_ref, out_ref):
    out_ref[...] = jnp.broadcast_to(clean_ref[...][None], (G, SEQ, D))


_tc_broadcast = pl.pallas_call(
    _bcast_body,
    grid=(BATCH // G,),
    in_specs=[pl.BlockSpec((SEQ, D), lambda i: (0, 0))],
    out_specs=pl.BlockSpec((G, SEQ, D), lambda i: (i, 0, 0)),
    out_shape=jax.ShapeDtypeStruct((BATCH, SEQ, D), jnp.float32),
)

_mesh = plsc.VectorSubcoreMesh(core_axis_name="c", subcore_axis_name="s")


def _rebuild_machinery(pe_v, x_v, outbuf_v, idx_v, lane):
    def copy_rows():
        def sstep(i, carry):
            s = i * 2
            ivv = idx_v[pl.ds(s, L)]
            for k in range(2):
                src = ivv[k] * D
                dst = (s + k) * D
                for u in range(D // L):
                    outbuf_v[pl.ds(dst + u * L, L)] = (
                        pe_v[pl.ds(src + u * L, L)]
                    )
            return carry

        lax.fori_loop(0, SEQ // 2, sstep, 0)

    def row_pad_acc(rb):
        acc = x_v[pl.ds(rb, L)] == PAD
        for w in range(1, NWIN):
            acc = jnp.logical_or(acc, x_v[pl.ds(rb + w * L, L)] == PAD)
        return jnp.logical_or(acc, x_v[pl.ds(rb + SEQ - L, L)] == PAD)

    def build_row(b):
        rb = b * SEQ
        run = jnp.int32(0)
        for w in range(NWIN):
            v = x_v[pl.ds(rb + w * L, L)]
            m = v != PAD
            c = plsc.cumsum(m.astype(jnp.int32))
            idx_v[pl.ds(w * L, L)] = jnp.where(m, run + c, 0)
            run = run + c[L - 1]
        v = x_v[pl.ds(rb + SEQ - L, L)]
        m = v != PAD
        c = plsc.cumsum(m.astype(jnp.int32))
        idx_v[pl.ds(SEQ - L, L)] = jnp.where(m, run - c[7] + c, 0)
        copy_rows()

    def scan_groups(on_dirty_row):
        def handle_group(g, carry):
            gb = g * L * SEQ
            gacc = row_pad_acc(gb)
            for r in range(1, L):
                gacc = jnp.logical_or(gacc, row_pad_acc(gb + r * SEQ))
            ganyz = lax.reduce_or(gacc, axes=(0,))

            @pl.when(ganyz)
            def _dirty_group():
                def hunt_row(r, carry2):
                    b = g * L + r
                    ranyz = lax.reduce_or(row_pad_acc(b * SEQ), axes=(0,))

                    @pl.when(ranyz)
                    def _fix():
                        on_dirty_row(b)

                    return carry2

                lax.fori_loop(0, L, hunt_row, 0)

            return carry

        lax.fori_loop(0, RPW // L, handle_group, 0)

    return build_row, scan_groups


@functools.partial(
    pl.kernel,
    out_type=(
        jax.ShapeDtypeStruct((NW * L,), jnp.int32),
        jax.ShapeDtypeStruct((NW * STAGE_K * ROW_WORDS,), jnp.float32),
    ),
    mesh=_mesh,
    compiler_params=pltpu.CompilerParams(needs_layout_passes=False),
    scratch_types=[
        pltpu.VMEM((PE_WORDS,), jnp.float32),
        pltpu.VMEM((RPW * SEQ,), jnp.int32),
        pltpu.VMEM((ROW_WORDS,), jnp.float32),
        pltpu.VMEM((SEQ + L,), jnp.int32),
        pltpu.VMEM((L,), jnp.int32),
        pltpu.SemaphoreType.DMA,
    ],
)
def _sc_scan(x_hbm, pe_hbm, meta_hbm, stage_hbm,
             pe_v, x_v, outbuf_v, idx_v, meta_v, sem):
    wid = lax.axis_index("s") * NC + lax.axis_index("c")

    stage_pe = pltpu.async_copy(pe_hbm, pe_v, sem)
    stage_x = pltpu.async_copy(
        x_hbm.at[pl.ds(wid * RPW * SEQ, RPW * SEQ)], x_v, sem
    )
    stage_pe.wait()
    stage_x.wait()

    lane = lax.iota(jnp.int32, L)
    meta_v[pl.ds(0, L)] = jnp.zeros((L,), jnp.int32)

    build_row, scan_groups = _rebuild_machinery(pe_v, x_v, outbuf_v, idx_v, lane)

    def stage_dirty_row(b):
        mv = meta_v[pl.ds(0, L)]
        cnt = mv[0]

        @pl.when(cnt < STAGE_K)
        def _stage():
            build_row(b)
            slot = (wid * STAGE_K + cnt) * ROW_WORDS
            pltpu.sync_copy(outbuf_v, stage_hbm.at[pl.ds(slot, ROW_WORDS)])

        mv2 = jnp.where(lane == cnt + 1, b, mv)
        meta_v[pl.ds(0, L)] = jnp.where(lane == 0, cnt + 1, mv2)

    scan_groups(stage_dirty_row)

    pltpu.sync_copy(meta_v, meta_hbm.at[pl.ds(wid * L, L)])


@functools.partial(
    pl.kernel,
    mesh=_mesh,
    compiler_params=pltpu.CompilerParams(needs_layout_passes=False),
    scratch_types=[
        pltpu.VMEM((PE_WORDS,), jnp.float32),
        pltpu.VMEM((RPW * SEQ,), jnp.int32),
        pltpu.VMEM((ROW_WORDS,), jnp.float32),
        pltpu.VMEM((SEQ + L,), jnp.int32),
        pltpu.VMEM((L,), jnp.int32),
        pltpu.SemaphoreType.DMA,
    ],
)
def _sc_patch(meta_hbm, stage_hbm, x_hbm, pe_hbm, out_hbm,
              pe_v, x_v, outbuf_v, idx_v, meta_v, sem):
    wid = lax.axis_index("s") * NC + lax.axis_index("c")
    base_row = wid * RPW

    pltpu.sync_copy(meta_hbm.at[pl.ds(wid * L, L)], meta_v)
    lane = lax.iota(jnp.int32, L)
    mv = meta_v[pl.ds(0, L)]
    cnt = mv[0]

    for i in range(STAGE_K):
        @pl.when(jnp.logical_and(i < cnt, cnt <= STAGE_K))
        def _patch(i=i):
            b = mv[i + 1]
            slot = (wid * STAGE_K + i) * ROW_WORDS
            out_off = (base_row + b) * ROW_WORDS
            pltpu.sync_copy(stage_hbm.at[pl.ds(slot, ROW_WORDS)],
                            out_hbm.at[pl.ds(out_off, ROW_WORDS)])

    @pl.when(cnt > STAGE_K)
    def _overflow():
        stage_pe = pltpu.async_copy(pe_hbm, pe_v, sem)
        stage_x = pltpu.async_copy(
            x_hbm.at[pl.ds(wid * RPW * SEQ, RPW * SEQ)], x_v, sem
        )
        stage_pe.wait()
        stage_x.wait()
        build_row, scan_groups = _rebuild_machinery(
            pe_v, x_v, outbuf_v, idx_v, lane
        )

        def fix_row(b):
            build_row(b)
            out_off = (base_row + b) * ROW_WORDS
            pltpu.sync_copy(outbuf_v, out_hbm.at[pl.ds(out_off, ROW_WORDS)])

        scan_groups(fix_row)


def kernel(x, pe):
    xf = x.reshape(-1).astype(jnp.int32)
    pef = pe.reshape(-1).astype(jnp.float32)
    base = _tc_broadcast(pe[1:].astype(jnp.float32))
    meta, stage = _sc_scan(xf, pef)
    out_ref = jax.new_ref(base.reshape(-1))
    _sc_patch(meta, stage, xf, pef, out_ref)
    return out_ref[...].reshape(BATCH, SEQ, D)


# speedup vs baseline: 1.2264x; 1.2264x over previous
import functools

import jax
import jax.numpy as jnp
from jax import lax
from jax.experimental import pallas as pl
from jax.experimental.pallas import tpu as pltpu
from jax.experimental.pallas import tpu_sc as plsc

PAD = 0
BATCH = 4096
SEQ = 200
D = 128
PE_ROWS = 201
NC, NS, L = 2, 16, 16
NW = NC * NS
RPW = BATCH // NW
ROW_WORDS = SEQ * D
PE_WORDS = PE_ROWS * D
NWIN = SEQ // L
STAGE_K = L - 1
G = 128


def _bcast_body(clean_ref, out_ref):
    out_ref[...] = jnp.broadcast_to(clean_ref[...][None], (G, SEQ, D))


_tc_broadcast = pl.pallas_call(
    _bcast_body,
    grid=(BATCH // G,),
    in_specs=[pl.BlockSpec((SEQ, D), lambda i: (0, 0))],
    out_specs=pl.BlockSpec((G, SEQ, D), lambda i: (i, 0, 0)),
    out_shape=jax.ShapeDtypeStruct((BATCH, SEQ, D), jnp.float32),
)

_mesh = plsc.VectorSubcoreMesh(core_axis_name="c", subcore_axis_name="s")


def _rebuild_machinery(pe_v, x_v, outbuf_v, idx_v, lane):
    def copy_rows():
        def sstep(i, carry):
            s = i * 2
            ivv = idx_v[pl.ds(s, L)]
            for k in range(2):
                src = ivv[k] * D
                dst = (s + k) * D
                for u in range(D // L):
                    outbuf_v[pl.ds(dst + u * L, L)] = (
                        pe_v[pl.ds(src + u * L, L)]
                    )
            return carry

        lax.fori_loop(0, SEQ // 2, sstep, 0)

    def row_pad_acc(rb):
        acc = x_v[pl.ds(rb, L)] == PAD
        for w in range(1, NWIN):
            acc = jnp.logical_or(acc, x_v[pl.ds(rb + w * L, L)] == PAD)
        return jnp.logical_or(acc, x_v[pl.ds(rb + SEQ - L, L)] == PAD)

    def build_row(b):
        rb = b * SEQ
        run = jnp.int32(0)
        for w in range(NWIN):
            v = x_v[pl.ds(rb + w * L, L)]
            m = v != PAD
            c = plsc.cumsum(m.astype(jnp.int32))
            idx_v[pl.ds(w * L, L)] = jnp.where(m, run + c, 0)
            run = run + c[L - 1]
        v = x_v[pl.ds(rb + SEQ - L, L)]
        m = v != PAD
        c = plsc.cumsum(m.astype(jnp.int32))
        idx_v[pl.ds(SEQ - L, L)] = jnp.where(m, run - c[7] + c, 0)
        copy_rows()

    def scan_groups(on_dirty_row):
        def handle_group(g, carry):
            gb = g * L * SEQ
            gacc = row_pad_acc(gb)
            for r in range(1, L):
                gacc = jnp.logical_or(gacc, row_pad_acc(gb + r * SEQ))
            ganyz = lax.reduce_or(gacc, axes=(0,))

            @pl.when(ganyz)
            def _dirty_group():
                def hunt_row(r, carry2):
                    b = g * L + r
                    ranyz = lax.reduce_or(row_pad_acc(b * SEQ), axes=(0,))

                    @pl.when(ranyz)
                    def _fix():
                        on_dirty_row(b)

                    return carry2

                lax.fori_loop(0, L, hunt_row, 0)

            return carry

        lax.fori_loop(0, RPW // L, handle_group, 0)

    return build_row, scan_groups


@functools.partial(
    pl.kernel,
    out_type=(
        jax.ShapeDtypeStruct((NW * L,), jnp.int32),
        jax.ShapeDtypeStruct((NW * STAGE_K * ROW_WORDS,), jnp.float32),
    ),
    mesh=_mesh,
    compiler_params=pltpu.CompilerParams(needs_layout_passes=False),
    scratch_types=[
        pltpu.VMEM((PE_WORDS,), jnp.float32),
        pltpu.VMEM((RPW * SEQ,), jnp.int32),
        pltpu.VMEM((ROW_WORDS,), jnp.float32),
        pltpu.VMEM((SEQ + L,), jnp.int32),
        pltpu.VMEM((L,), jnp.int32),
        pltpu.SemaphoreType.DMA,
    ],
)
def _sc_scan(x_hbm, pe_hbm, meta_hbm, stage_hbm,
             pe_v, x_v, outbuf_v, idx_v, meta_v, sem):
    wid = lax.axis_index("s") * NC + lax.axis_index("c")

    stage_pe = pltpu.async_copy(pe_hbm, pe_v, sem)
    stage_x = pltpu.async_copy(
        x_hbm.at[pl.ds(wid * RPW * SEQ, RPW * SEQ)], x_v, sem
    )
    stage_pe.wait()
    stage_x.wait()

    lane = lax.iota(jnp.int32, L)
    meta_v[pl.ds(0, L)] = jnp.zeros((L,), jnp.int32)

    build_row, scan_groups = _rebuild_machinery(pe_v, x_v, outbuf_v, idx_v, lane)

    def stage_dirty_row(b):
        mv = meta_v[pl.ds(0, L)]
        cnt = mv[0]

        @pl.when(cnt < STAGE_K)
        def _stage():
            build_row(b)
            slot = (wid * STAGE_K + cnt) * ROW_WORDS
            pltpu.sync_copy(outbuf_v, stage_hbm.at[pl.ds(slot, ROW_WORDS)])

        mv2 = jnp.where(lane == cnt + 1, b, mv)
        meta_v[pl.ds(0, L)] = jnp.where(lane == 0, cnt + 1, mv2)

    scan_groups(stage_dirty_row)

    pltpu.sync_copy(meta_v, meta_hbm.at[pl.ds(wid * L, L)])


@functools.partial(
    pl.kernel,
    mesh=_mesh,
    compiler_params=pltpu.CompilerParams(needs_layout_passes=False),
    scratch_types=[
        pltpu.VMEM((PE_WORDS,), jnp.float32),
        pltpu.VMEM((RPW * SEQ,), jnp.int32),
        pltpu.VMEM((ROW_WORDS,), jnp.float32),
        pltpu.VMEM((SEQ + L,), jnp.int32),
        pltpu.VMEM((L,), jnp.int32),
        pltpu.SemaphoreType.DMA,
    ],
)
def _sc_patch(meta_hbm, stage_hbm, x_hbm, pe_hbm, out_hbm,
              pe_v, x_v, outbuf_v, idx_v, meta_v, sem):
    wid = lax.axis_index("s") * NC + lax.axis_index("c")
    base_row = wid * RPW

    pltpu.sync_copy(meta_hbm.at[pl.ds(wid * L, L)], meta_v)
    lane = lax.iota(jnp.int32, L)
    mv = meta_v[pl.ds(0, L)]
    cnt = mv[0]

    for i in range(STAGE_K):
        @pl.when(jnp.logical_and(i < cnt, cnt <= STAGE_K))
        def _patch(i=i):
            b = mv[i + 1]
            slot = (wid * STAGE_K + i) * ROW_WORDS
            pltpu.sync_copy(stage_hbm.at[pl.ds(slot, ROW_WORDS)], outbuf_v)
            out_off = (base_row + b) * ROW_WORDS
            pltpu.sync_copy(outbuf_v, out_hbm.at[pl.ds(out_off, ROW_WORDS)])

    @pl.when(cnt > STAGE_K)
    def _overflow():
        stage_pe = pltpu.async_copy(pe_hbm, pe_v, sem)
        stage_x = pltpu.async_copy(
            x_hbm.at[pl.ds(wid * RPW * SEQ, RPW * SEQ)], x_v, sem
        )
        stage_pe.wait()
        stage_x.wait()
        build_row, scan_groups = _rebuild_machinery(
            pe_v, x_v, outbuf_v, idx_v, lane
        )

        def fix_row(b):
            build_row(b)
            out_off = (base_row + b) * ROW_WORDS
            pltpu.sync_copy(outbuf_v, out_hbm.at[pl.ds(out_off, ROW_WORDS)])

        scan_groups(fix_row)


def kernel(x, pe):
    xf = x.reshape(-1).astype(jnp.int32)
    pef = pe.reshape(-1).astype(jnp.float32)
    base = _tc_broadcast(pe[1:].astype(jnp.float32))
    meta, stage = _sc_scan(xf, pef)
    out_ref = jax.new_ref(base.reshape(-1))
    _sc_patch(meta, stage, xf, pef, out_ref)
    return out_ref[...].reshape(BATCH, SEQ, D)
